# baseline (speedup 1.0000x reference)
import functools

import jax
import jax.numpy as jnp
from jax import lax
from jax.experimental import pallas as pl
from jax.experimental.pallas import tpu as pltpu
from jax.experimental.pallas import tpu_sc as plsc

_NC = 2
_NS = 16
_NW = _NC * _NS
_L = 16

_A = 4096
_P = 26
_S = 20
_C = _P * _S
_B = _A * _C
_D = 64
_APW = _A // _NW

_ZROW = 42
_VPAD = 48


@functools.partial(
    pl.kernel,
    out_type=(
        jax.ShapeDtypeStruct((_A, _P, _S, _D), jnp.float32),
        jax.ShapeDtypeStruct((_B,), jnp.int32),
    ),
    mesh=plsc.VectorSubcoreMesh(core_axis_name="c", subcore_axis_name="s"),
    compiler_params=pltpu.CompilerParams(use_tc_tiling_on_sc=False),
    scratch_types=[
        pltpu.VMEM_SHARED((_VPAD, _D), jnp.float32),
        pltpu.VMEM((_VPAD, _D), jnp.float32),
        pltpu.VMEM((_C,), jnp.int32),
        pltpu.VMEM((2, _P, 32), jnp.int32),
        pltpu.VMEM((2, _C), jnp.int32),
        pltpu.VMEM((2, _P, _S, _D), jnp.float32),
        pltpu.SemaphoreType.DMA,
        pltpu.SemaphoreType.DMA,
        pltpu.SemaphoreType.DMA,
        pltpu.SemaphoreType.DMA,
        pltpu.SemaphoreType.DMA,
        pltpu.SemaphoreType.DMA,
    ],
)
def _emb_lookup(table_hbm, labels_hbm, out_hbm, mask_hbm,
                table_sp, table_v, raw_v, idx_v, mask_v, rows_v,
                gsem0, gsem1, osem0, osem1, msem0, msem1):
    cid = lax.axis_index("c")
    sid = lax.axis_index("s")
    wid = sid * _NC + cid
    a0 = wid * _APW

    @pl.when(sid == 0)
    def _():
        pltpu.sync_copy(table_hbm, table_v)
        pltpu.sync_copy(table_v, table_sp)

    plsc.subcore_barrier()

    gsems = (gsem0, gsem1)
    osems = (osem0, osem1)
    msems = (msem0, msem1)

    def preprocess_group(src_off, p, col, b):
        lab = raw_v[pl.ds(src_off, _L)]
        is_pad = lab == jnp.full((_L,), -1, jnp.int32)
        mask_v[b, pl.ds(src_off, _L)] = jnp.where(
            is_pad, jnp.zeros((_L,), jnp.int32), jnp.ones((_L,), jnp.int32))
        idx_v[b, p, pl.ds(col, _L)] = jnp.where(
            is_pad, jnp.full((_L,), _ZROW, jnp.int32), lab)

    def load_and_preprocess(i, b):
        pltpu.sync_copy(labels_hbm.at[pl.ds((a0 + i) * _C, _C)], raw_v)

        def plane_body(p, carry):
            preprocess_group(p * _S, p, 0, b)
            preprocess_group(p * _S + (_S - _L), p, _S - _L, b)
            return carry

        lax.fori_loop(0, _P, plane_body, 0)

    def fire_gather(b):
        for p in range(_P):
            pltpu.async_copy(
                table_sp.at[idx_v.at[b].at[p].at[pl.ds(0, _S)]],
                rows_v.at[b].at[p], gsems[b])

    def wait_gather(b):
        for p in range(_P):
            pltpu.make_async_copy(
                table_sp.at[idx_v.at[b].at[p].at[pl.ds(0, _S)]],
                rows_v.at[b].at[p], gsems[b]).wait()

    def fire_out(i, b):
        pltpu.async_copy(rows_v.at[b], out_hbm.at[a0 + i], osems[b])
        pltpu.async_copy(mask_v.at[b],
                         mask_hbm.at[pl.ds((a0 + i) * _C, _C)], msems[b])

    def wait_out(i, b):
        pltpu.make_async_copy(rows_v.at[b], out_hbm.at[a0 + i],
                              osems[b]).wait()
        pltpu.make_async_copy(mask_v.at[b],
                              mask_hbm.at[pl.ds((a0 + i) * _C, _C)],
                              msems[b]).wait()

    def step(i, b):
        b2 = 1 - b
        wait_gather(b2)
        fire_out(i - 1, b2)

        @pl.when(i >= 2)
        def _():
            wait_out(i - 2, b)

        load_and_preprocess(i, b)
        fire_gather(b)

    load_and_preprocess(0, 0)
    fire_gather(0)

    def pair_body(g, carry):
        step(2 * g + 1, 1)
        step(2 * g + 2, 0)
        return carry

    lax.fori_loop(0, _APW // 2 - 1, pair_body, 0)
    step(_APW - 1, 1)
    wait_gather(1)
    fire_out(_APW - 1, 1)
    wait_out(_APW - 2, 0)
    wait_out(_APW - 1, 1)


def kernel(output, action_emb):
    labels = output[0].reshape(_B)
    table = jnp.concatenate(
        [action_emb, jnp.zeros((_VPAD - action_emb.shape[0], _D), jnp.float32)])
    emb, mask_flat = _emb_lookup(table, labels)
    return (emb, mask_flat.reshape(_A, _P, _S))
